# baseline (speedup 1.0000x reference)
import functools

import jax
import jax.numpy as jnp
from jax import lax
from jax.experimental import pallas as pl
from jax.experimental.pallas import tpu as pltpu
from jax.experimental.pallas import tpu_sc as plsc

_NC = 2
_NS = 16
_LANES = 16


@functools.lru_cache(maxsize=None)
def _make_rpe(S: int, T: int, D: int):
    CH = T // _NS
    RPT = S // (_NC * _NS)
    WL = S // 2
    WR = S - WL
    WIN = WL + RPT - 1
    assert T % _NS == 0 and S % (_NC * _NS) == 0 and D % _LANES == 0

    mesh = plsc.VectorSubcoreMesh(
        core_axis_name="c", subcore_axis_name="s",
        num_cores=_NC, num_subcores=_NS,
    )

    @functools.partial(
        pl.kernel,
        out_type=jax.ShapeDtypeStruct((S, S, D), jnp.float32),
        mesh=mesh,
        scratch_types=[
            pltpu.VMEM_SHARED((T, D), jnp.float32),
            pltpu.VMEM((CH, D), jnp.float32),
            pltpu.VMEM((CH, D), jnp.float32),
            pltpu.VMEM((WIN, D), jnp.float32),
            pltpu.SemaphoreType.DMA,
            pltpu.SemaphoreType.DMA,
        ],
    )
    def rpe(table_hbm, out_hbm, rev_sp, buf_in, buf_out, win, sem_l, sem_r):
        c = lax.axis_index("c")
        s = lax.axis_index("s")

        src_lo = T - (s + 1) * CH
        pltpu.sync_copy(table_hbm.at[pl.ds(src_lo, CH)], buf_in)

        def rev_row(r, carry):
            for k in range(D // _LANES):
                buf_out[CH - 1 - r, pl.ds(k * _LANES, _LANES)] = (
                    buf_in[r, pl.ds(k * _LANES, _LANES)]
                )
            return carry

        lax.fori_loop(0, CH, rev_row, 0)
        pltpu.sync_copy(buf_out, rev_sp.at[pl.ds(s * CH, CH)])
        plsc.subcore_barrier()

        wid = s * _NC + c
        o_min = T - S - (wid * RPT + RPT - 1)
        pltpu.sync_copy(rev_sp.at[pl.ds(o_min, WIN)], win)

        DEPTH = 3

        def issue(k):
            i = wid * RPT + k
            o = T - S - i
            pltpu.async_copy(
                win.at[pl.ds(RPT - 1 - k, WL)],
                out_hbm.at[i, pl.ds(0, WL)],
                sem_l,
            )
            pltpu.async_copy(
                rev_sp.at[pl.ds(o + WL, WR)],
                out_hbm.at[i, pl.ds(WL, WR)],
                sem_r,
            )

        def drain_one():
            pltpu.make_async_copy(
                win.at[pl.ds(0, WL)], out_hbm.at[0, pl.ds(0, WL)], sem_l
            ).wait()
            pltpu.make_async_copy(
                rev_sp.at[pl.ds(0, WR)], out_hbm.at[0, pl.ds(WL, WR)], sem_r
            ).wait()

        for k in range(DEPTH):
            issue(k)

        def out_row(k, carry):
            issue(k)
            drain_one()
            return carry

        lax.fori_loop(DEPTH, RPT, out_row, 0)
        for _ in range(DEPTH):
            drain_one()

    return rpe


def kernel(x, relative_embeddings):
    S = x.shape[1]
    T, D = relative_embeddings.shape
    return _make_rpe(S, T, D)(relative_embeddings)


# speedup vs baseline: 1.0384x; 1.0384x over previous
import functools

import jax
import jax.numpy as jnp
from jax import lax
from jax.experimental import pallas as pl
from jax.experimental.pallas import tpu as pltpu
from jax.experimental.pallas import tpu_sc as plsc

_NC = 2
_NS = 16
_LANES = 16


@functools.lru_cache(maxsize=None)
def _make_rpe(S: int, T: int, D: int):
    CH = T // _NS
    RPT = S // (_NC * _NS)
    WL = S // 2
    WR = S - WL
    WIN = WL + RPT - 1
    assert T % _NS == 0 and S % (_NC * _NS) == 0 and D % _LANES == 0

    mesh = plsc.VectorSubcoreMesh(
        core_axis_name="c", subcore_axis_name="s",
        num_cores=_NC, num_subcores=_NS,
    )

    @functools.partial(
        pl.kernel,
        out_type=jax.ShapeDtypeStruct((S, S, D), jnp.float32),
        mesh=mesh,
        scratch_types=[
            pltpu.VMEM_SHARED((T, D), jnp.float32),
            pltpu.VMEM((CH, D), jnp.float32),
            pltpu.VMEM((CH, D), jnp.float32),
            pltpu.VMEM((WIN, D), jnp.float32),
            pltpu.SemaphoreType.DMA,
            pltpu.SemaphoreType.DMA,
            pltpu.SemaphoreType.DMA,
        ],
    )
    def rpe(table_hbm, out_hbm, rev_sp, buf_in, buf_out, win,
            sem_l, sem_r, sem_w):
        c = lax.axis_index("c")
        s = lax.axis_index("s")

        src_lo = T - (s + 1) * CH
        pltpu.sync_copy(table_hbm.at[pl.ds(src_lo, CH)], buf_in)

        def rev_row(r, carry):
            for k in range(D // _LANES):
                buf_out[CH - 1 - r, pl.ds(k * _LANES, _LANES)] = (
                    buf_in[r, pl.ds(k * _LANES, _LANES)]
                )
            return carry

        lax.fori_loop(0, CH, rev_row, 0)
        pltpu.sync_copy(buf_out, rev_sp.at[pl.ds(s * CH, CH)])
        plsc.subcore_barrier()

        wid = s * _NC + c
        o_min = T - S - (wid * RPT + RPT - 1)
        wfill = pltpu.async_copy(rev_sp.at[pl.ds(o_min, WIN)], win, sem_w)

        PRE = 2

        def issue_right(k):
            i = wid * RPT + k
            o = T - S - i
            pltpu.async_copy(
                rev_sp.at[pl.ds(o + WL, WR)],
                out_hbm.at[i, pl.ds(WL, WR)],
                sem_r,
            )

        for k in range(PRE):
            issue_right(k)
        wfill.wait()

        def out_row(k, carry):
            i = wid * RPT + k
            cp_l = pltpu.async_copy(
                win.at[pl.ds(RPT - 1 - k, WL)],
                out_hbm.at[i, pl.ds(0, WL)],
                sem_l,
            )

            @pl.when(k < RPT - PRE)
            def _():
                issue_right(k + PRE)

            pltpu.make_async_copy(
                rev_sp.at[pl.ds(0, WR)], out_hbm.at[0, pl.ds(WL, WR)], sem_r
            ).wait()
            cp_l.wait()
            return carry

        lax.fori_loop(0, RPT, out_row, 0)

    return rpe


def kernel(x, relative_embeddings):
    S = x.shape[1]
    T, D = relative_embeddings.shape
    return _make_rpe(S, T, D)(relative_embeddings)
